# baseline (speedup 1.0000x reference)
import functools

import jax
import jax.numpy as jnp
from jax import lax
from jax.experimental import pallas as pl
from jax.experimental.pallas import tpu as pltpu
from jax.experimental.pallas import tpu_sc as plsc

N = 10000
E = 160000
D = 256
H = 128
NC = 2
NS = 16
K = 128
EP = 163840
NCHUNK = EP // K
CHT_MM = NCHUNK // NS
CHT2 = CHT_MM // 2
CHT_HIST = NCHUNK // (NC * NS)
NP = 10112
RPT = NP // NS
NFULL = RPT // K
REM = RPT % K
EPS = 1e-7

_mesh = plsc.VectorSubcoreMesh(core_axis_name="c", subcore_axis_name="s")


def _zero_rows(buf, nrows):
    zeros16 = jnp.zeros((16,), jnp.float32)

    @pl.loop(0, nrows)
    def _(r):
        for i in range(H // 16):
            buf[r, pl.ds(i * 16, 16)] = zeros16


def _make_deg_kernel(W):
    @functools.partial(
        pl.kernel,
        out_type=jax.ShapeDtypeStruct((NC, NP, W), jnp.float32),
        mesh=_mesh,
        scratch_types=[
            pltpu.VMEM((CHT_HIST, K), jnp.int32),
            pltpu.VMEM((K, W), jnp.float32),
            pltpu.VMEM((K, W), jnp.float32),
            pltpu.VMEM_SHARED((NP, W), jnp.float32),
        ],
    )
    def deg_kernel(row_hbm, out_hbm, row_v, ones_v, stage_v, acc):
        c = lax.axis_index("c")
        s = lax.axis_index("s")
        w = c * NS + s
        pltpu.sync_copy(row_hbm.at[pl.ds(w * CHT_HIST, CHT_HIST)], row_v)

        ones16 = jnp.ones((16,), jnp.float32)
        zeros16 = jnp.zeros((16,), jnp.float32)

        @pl.loop(0, K)
        def _(r):
            for i in range(W // 16):
                ones_v[r, pl.ds(i * 16, 16)] = ones16
                stage_v[r, pl.ds(i * 16, 16)] = zeros16

        rbase = s * RPT
        for q in range(NFULL):
            pltpu.sync_copy(stage_v, acc.at[pl.ds(rbase + q * K, K)])
        pltpu.sync_copy(stage_v.at[pl.ds(0, REM)],
                        acc.at[pl.ds(rbase + NFULL * K, REM)])
        plsc.subcore_barrier()

        @pl.loop(0, CHT_HIST)
        def _(j):
            pltpu.sync_copy(ones_v, acc.at[row_v.at[j]], add=True)

        plsc.subcore_barrier()

        for q in range(NFULL):
            pltpu.sync_copy(acc.at[pl.ds(rbase + q * K, K)], stage_v)
            pltpu.sync_copy(stage_v, out_hbm.at[c, pl.ds(rbase + q * K, K)])
        pltpu.sync_copy(acc.at[pl.ds(rbase + NFULL * K, REM)],
                        stage_v.at[pl.ds(0, REM)])
        pltpu.sync_copy(stage_v.at[pl.ds(0, REM)],
                        out_hbm.at[c, pl.ds(rbase + NFULL * K, REM)])

    return deg_kernel


_deg_kernel = _make_deg_kernel(H)


@functools.partial(
    pl.kernel,
    out_type=jax.ShapeDtypeStruct((NC, NP, H), jnp.float32),
    mesh=_mesh,
    scratch_types=[
        pltpu.VMEM((CHT2, K), jnp.int32),
        pltpu.VMEM((CHT2, K), jnp.int32),
        pltpu.VMEM((K, H), jnp.float32),
        pltpu.VMEM((K, H), jnp.float32),
        pltpu.VMEM_SHARED((NP, H), jnp.float32),
        pltpu.SemaphoreType.DMA,
        pltpu.SemaphoreType.DMA,
        pltpu.SemaphoreType.DMA,
        pltpu.SemaphoreType.DMA,
    ],
)
def _spmm_kernel(z_hbm, row_hbm, col_hbm, out_hbm,
                 row_v, gidx_v, buf0, buf1, acc, sem0, sem1, ssem0, ssem1):
    c = lax.axis_index("c")
    s = lax.axis_index("s")

    base = c * NP

    _zero_rows(buf0, K)
    rbase = s * RPT
    for q in range(NFULL):
        pltpu.sync_copy(buf0, acc.at[pl.ds(rbase + q * K, K)])
    pltpu.sync_copy(buf0.at[pl.ds(0, REM)],
                    acc.at[pl.ds(rbase + NFULL * K, REM)])
    plsc.subcore_barrier()

    for half in range(2):
        cb = s * CHT_MM + half * CHT2
        pltpu.sync_copy(row_hbm.at[pl.ds(cb, CHT2)], row_v)
        pltpu.sync_copy(col_hbm.at[pl.ds(cb, CHT2)], gidx_v)

        @pl.loop(0, CHT2)
        def _(j):
            for i in range(K // 16):
                sl = pl.ds(i * 16, 16)
                gidx_v[j, sl] = gidx_v[j, sl] + base

        bufs = ((buf0, sem0, ssem0), (buf1, sem1, ssem1))
        pltpu.async_copy(z_hbm.at[gidx_v.at[0]], buf0, sem0)
        pltpu.async_copy(z_hbm.at[gidx_v.at[1]], buf1, sem1)

        @pl.loop(0, CHT2, step=2)
        def _(g):
            for b, (buf, sem, ssem) in enumerate(bufs):
                j = g + b
                pltpu.make_async_copy(z_hbm.at[gidx_v.at[j]], buf, sem).wait()
                pltpu.async_copy(buf, acc.at[row_v.at[j]], ssem, add=True)
            for b, (buf, sem, ssem) in enumerate(bufs):
                j = g + b
                pltpu.make_async_copy(buf, acc.at[row_v.at[j]], ssem).wait()
                jn = j + 2

                @pl.when(jn < CHT2)
                def _():
                    pltpu.async_copy(z_hbm.at[gidx_v.at[jn]], buf, sem)

    plsc.subcore_barrier()

    for q in range(NFULL):
        pltpu.sync_copy(acc.at[pl.ds(rbase + q * K, K)], buf0)
        pltpu.sync_copy(buf0, out_hbm.at[c, pl.ds(rbase + q * K, K)])
    pltpu.sync_copy(acc.at[pl.ds(rbase + NFULL * K, REM)],
                    buf0.at[pl.ds(0, REM)])
    pltpu.sync_copy(buf0.at[pl.ds(0, REM)],
                    out_hbm.at[c, pl.ds(rbase + NFULL * K, REM)])


def _rsqrt_deg(deg_ref):
    deg = deg_ref[0] + deg_ref[1]
    return lax.rsqrt(deg + EPS)


def _scale_in_body(deg_ref, x_ref, out_ref):
    sinv = _rsqrt_deg(deg_ref)
    out_ref[0] = sinv * x_ref[:, :H]
    out_ref[1] = sinv * x_ref[:, H:]


def _scale_mid_body(deg_ref, z_ref, out_ref):
    deg = deg_ref[0] + deg_ref[1]
    s2 = 1.0 / (deg + EPS)
    out_ref[0] = z_ref[0] * s2
    out_ref[1] = z_ref[1] * s2


def _final_body(deg_ref, z_ref, x_ref, out_ref):
    sinv = _rsqrt_deg(deg_ref)[:N]
    out_ref[:, :H] = x_ref[:, :H] + sinv * z_ref[0, :N, :]
    out_ref[:, H:] = x_ref[:, H:] + sinv * z_ref[1, :N, :]


def kernel(x, edge_index):
    row = edge_index[0].astype(jnp.int32)
    col = edge_index[1].astype(jnp.int32)
    pad = EP - E
    rowp = jnp.concatenate([row, jnp.full((pad,), N, jnp.int32)]).reshape(NCHUNK, K)
    colp = jnp.concatenate([col, jnp.full((pad,), N, jnp.int32)]).reshape(NCHUNK, K)
    xp = jnp.zeros((NP, D), x.dtype).at[:N].set(x)

    deg2 = _deg_kernel(rowp)

    z0 = pl.pallas_call(
        _scale_in_body,
        out_shape=jax.ShapeDtypeStruct((NC, NP, H), jnp.float32),
    )(deg2, xp)

    z1 = _spmm_kernel(z0.reshape(NC * NP, H), rowp, colp)

    z1s = pl.pallas_call(
        _scale_mid_body,
        out_shape=jax.ShapeDtypeStruct((NC, NP, H), jnp.float32),
    )(deg2, z1)

    z2 = _spmm_kernel(z1s.reshape(NC * NP, H), rowp, colp)

    out = pl.pallas_call(
        _final_body,
        out_shape=jax.ShapeDtypeStruct((N, D), jnp.float32),
    )(deg2, z2, x)
    return out


# speedup vs baseline: 1.0116x; 1.0116x over previous
import functools

import jax
import jax.numpy as jnp
from jax import lax
from jax.experimental import pallas as pl
from jax.experimental.pallas import tpu as pltpu
from jax.experimental.pallas import tpu_sc as plsc

N = 10000
E = 160000
D = 256
H = 128
NC = 2
NS = 16
K = 128
EP = 163840
NCHUNK = EP // K
CHT_MM = NCHUNK // NS
CHT2 = CHT_MM // 2
CHT_HIST = NCHUNK // (NC * NS)
NP = 10112
RPT = NP // NS
NFULL = RPT // K
REM = RPT % K
EPS = 1e-7

_mesh = plsc.VectorSubcoreMesh(core_axis_name="c", subcore_axis_name="s")


def _make_deg_kernel(W):
    @functools.partial(
        pl.kernel,
        out_type=jax.ShapeDtypeStruct((NC, NP, W), jnp.float32),
        mesh=_mesh,
        scratch_types=[
            pltpu.VMEM((CHT_HIST, K), jnp.int32),
            pltpu.VMEM((K, W), jnp.float32),
            pltpu.VMEM((K, W), jnp.float32),
            pltpu.VMEM_SHARED((NP, W), jnp.float32),
        ],
    )
    def deg_kernel(row_hbm, const_hbm, out_hbm, row_v, ones_v, stage_v, acc):
        c = lax.axis_index("c")
        s = lax.axis_index("s")
        w = c * NS + s
        pltpu.sync_copy(row_hbm.at[pl.ds(w * CHT_HIST, CHT_HIST)], row_v)
        pltpu.sync_copy(const_hbm.at[0], ones_v)
        pltpu.sync_copy(const_hbm.at[1], stage_v)

        rbase = s * RPT
        for q in range(NFULL):
            pltpu.sync_copy(stage_v, acc.at[pl.ds(rbase + q * K, K)])
        pltpu.sync_copy(stage_v.at[pl.ds(0, REM)],
                        acc.at[pl.ds(rbase + NFULL * K, REM)])
        plsc.subcore_barrier()

        @pl.loop(0, CHT_HIST)
        def _(j):
            pltpu.sync_copy(ones_v, acc.at[row_v.at[j]], add=True)

        plsc.subcore_barrier()

        for q in range(NFULL):
            pltpu.sync_copy(acc.at[pl.ds(rbase + q * K, K)], stage_v)
            pltpu.sync_copy(stage_v, out_hbm.at[c, pl.ds(rbase + q * K, K)])
        pltpu.sync_copy(acc.at[pl.ds(rbase + NFULL * K, REM)],
                        stage_v.at[pl.ds(0, REM)])
        pltpu.sync_copy(stage_v.at[pl.ds(0, REM)],
                        out_hbm.at[c, pl.ds(rbase + NFULL * K, REM)])

    return deg_kernel


_deg_kernel = _make_deg_kernel(H)


def _make_spmm_kernel(scaled):
    def body(z_hbm, row_hbm, col_hbm, const_hbm, *rest):
        if scaled:
            (s2_hbm, out_hbm, row_v, gidx_v, buf0, buf1,
             acc, sem0, sem1) = rest
        else:
            out_hbm, row_v, gidx_v, buf0, buf1, acc, sem0, sem1 = rest
        c = lax.axis_index("c")
        s = lax.axis_index("s")

        base = c * NP

        pltpu.sync_copy(const_hbm.at[1], buf0)
        rbase = s * RPT
        for q in range(NFULL):
            pltpu.sync_copy(buf0, acc.at[pl.ds(rbase + q * K, K)])
        pltpu.sync_copy(buf0.at[pl.ds(0, REM)],
                        acc.at[pl.ds(rbase + NFULL * K, REM)])
        plsc.subcore_barrier()

        for half in range(2):
            cb = s * CHT_MM + half * CHT2
            pltpu.sync_copy(row_hbm.at[pl.ds(cb, CHT2)], row_v)
            pltpu.sync_copy(col_hbm.at[pl.ds(cb, CHT2)], gidx_v)

            @pl.loop(0, CHT2)
            def _(j):
                for i in range(K // 16):
                    sl = pl.ds(i * 16, 16)
                    gidx_v[j, sl] = gidx_v[j, sl] + base

            pltpu.async_copy(z_hbm.at[gidx_v.at[0]], buf0, sem0)
            pltpu.async_copy(z_hbm.at[gidx_v.at[1]], buf1, sem1)

            @pl.loop(0, CHT2, step=2)
            def _(g):
                for b, (buf, sem) in enumerate(((buf0, sem0), (buf1, sem1))):
                    j = g + b
                    pltpu.make_async_copy(z_hbm.at[gidx_v.at[j]], buf,
                                          sem).wait()
                    pltpu.sync_copy(buf, acc.at[row_v.at[j]], add=True)
                    jn = j + 2

                    @pl.when(jn < CHT2)
                    def _():
                        pltpu.async_copy(z_hbm.at[gidx_v.at[jn]], buf, sem)

        plsc.subcore_barrier()

        def flush(off, nrows, src_slice):
            pltpu.sync_copy(acc.at[pl.ds(off, nrows)], src_slice)
            if scaled:
                pltpu.sync_copy(s2_hbm.at[pl.ds(off, nrows)],
                                buf1.at[pl.ds(0, nrows)])

                @pl.loop(0, nrows)
                def _(r):
                    for i in range(H // 16):
                        sl = pl.ds(i * 16, 16)
                        buf0[r, sl] = buf0[r, sl] * buf1[r, sl]
            pltpu.sync_copy(src_slice, out_hbm.at[c, pl.ds(off, nrows)])

        for q in range(NFULL):
            flush(rbase + q * K, K, buf0)
        flush(rbase + NFULL * K, REM, buf0.at[pl.ds(0, REM)])

    scratch = [
        pltpu.VMEM((CHT2, K), jnp.int32),
        pltpu.VMEM((CHT2, K), jnp.int32),
        pltpu.VMEM((K, H), jnp.float32),
        pltpu.VMEM((K, H), jnp.float32),
        pltpu.VMEM_SHARED((NP, H), jnp.float32),
        pltpu.SemaphoreType.DMA,
        pltpu.SemaphoreType.DMA,
    ]
    return functools.partial(
        pl.kernel,
        out_type=jax.ShapeDtypeStruct((NC, NP, H), jnp.float32),
        mesh=_mesh,
        scratch_types=scratch,
    )(body)


_spmm_kernel = _make_spmm_kernel(scaled=False)
_spmm_scaled_kernel = _make_spmm_kernel(scaled=True)


def _rsqrt_deg(deg_ref):
    deg = deg_ref[0] + deg_ref[1]
    return lax.rsqrt(deg + EPS)


def _scale_in_body(deg_ref, x_ref, out_ref, s2_ref):
    deg = deg_ref[0] + deg_ref[1]
    sinv = lax.rsqrt(deg + EPS)
    out_ref[0] = sinv * x_ref[:, :H]
    out_ref[1] = sinv * x_ref[:, H:]
    s2_ref[...] = 1.0 / (deg + EPS)


def _final_body(deg_ref, z_ref, x_ref, out_ref):
    sinv = _rsqrt_deg(deg_ref)[:N]
    out_ref[:, :H] = x_ref[:, :H] + sinv * z_ref[0, :N, :]
    out_ref[:, H:] = x_ref[:, H:] + sinv * z_ref[1, :N, :]


def kernel(x, edge_index):
    row = edge_index[0].astype(jnp.int32)
    col = edge_index[1].astype(jnp.int32)
    pad = EP - E
    rowp = jnp.concatenate([row, jnp.full((pad,), N, jnp.int32)]).reshape(NCHUNK, K)
    colp = jnp.concatenate([col, jnp.full((pad,), N, jnp.int32)]).reshape(NCHUNK, K)
    xp = jnp.zeros((NP, D), x.dtype).at[:N].set(x)
    const = jnp.stack([jnp.ones((K, H), jnp.float32),
                       jnp.zeros((K, H), jnp.float32)])

    deg2 = _deg_kernel(rowp, const)

    z0, s2t = pl.pallas_call(
        _scale_in_body,
        out_shape=[jax.ShapeDtypeStruct((NC, NP, H), jnp.float32),
                   jax.ShapeDtypeStruct((NP, H), jnp.float32)],
    )(deg2, xp)

    z1s = _spmm_scaled_kernel(z0.reshape(NC * NP, H), rowp, colp, const, s2t)

    z2 = _spmm_kernel(z1s.reshape(NC * NP, H), rowp, colp, const)

    out = pl.pallas_call(
        _final_body,
        out_shape=jax.ShapeDtypeStruct((N, D), jnp.float32),
    )(deg2, z2, x)
    return out


# speedup vs baseline: 1.0492x; 1.0371x over previous
import functools

import jax
import jax.numpy as jnp
from jax import lax
from jax.experimental import pallas as pl
from jax.experimental.pallas import tpu as pltpu
from jax.experimental.pallas import tpu_sc as plsc

N = 10000
E = 160000
D = 256
H = 128
NC = 2
NS = 16
K = 128
EP = 163840
NCHUNK = EP // K
CHT_MM = NCHUNK // NS
CHT2 = CHT_MM // 2
CHT_HIST = NCHUNK // (NC * NS)
NP = 10112
RPT = NP // NS
NFULL = RPT // K
REM = RPT % K
EPS = 1e-7

_mesh = plsc.VectorSubcoreMesh(core_axis_name="c", subcore_axis_name="s")


def _make_deg_kernel(W):
    @functools.partial(
        pl.kernel,
        out_type=jax.ShapeDtypeStruct((NC, NP, W), jnp.float32),
        mesh=_mesh,
        scratch_types=[
            pltpu.VMEM((CHT_HIST, K), jnp.int32),
            pltpu.VMEM((K, W), jnp.float32),
            pltpu.VMEM((K, W), jnp.float32),
            pltpu.VMEM_SHARED((NP, W), jnp.float32),
        ],
    )
    def deg_kernel(row_hbm, const_hbm, out_hbm, row_v, ones_v, stage_v, acc):
        c = lax.axis_index("c")
        s = lax.axis_index("s")
        w = c * NS + s
        pltpu.sync_copy(row_hbm.at[pl.ds(w * CHT_HIST, CHT_HIST)], row_v)
        pltpu.sync_copy(const_hbm.at[0], ones_v)
        pltpu.sync_copy(const_hbm.at[1], stage_v)

        rbase = s * RPT
        for q in range(NFULL):
            pltpu.sync_copy(stage_v, acc.at[pl.ds(rbase + q * K, K)])
        pltpu.sync_copy(stage_v.at[pl.ds(0, REM)],
                        acc.at[pl.ds(rbase + NFULL * K, REM)])
        plsc.subcore_barrier()

        @pl.loop(0, CHT_HIST)
        def _(j):
            pltpu.sync_copy(ones_v, acc.at[row_v.at[j]], add=True)

        plsc.subcore_barrier()

        for q in range(NFULL):
            pltpu.sync_copy(acc.at[pl.ds(rbase + q * K, K)], stage_v)
            pltpu.sync_copy(stage_v, out_hbm.at[c, pl.ds(rbase + q * K, K)])
        pltpu.sync_copy(acc.at[pl.ds(rbase + NFULL * K, REM)],
                        stage_v.at[pl.ds(0, REM)])
        pltpu.sync_copy(stage_v.at[pl.ds(0, REM)],
                        out_hbm.at[c, pl.ds(rbase + NFULL * K, REM)])

    return deg_kernel


_deg_kernel = _make_deg_kernel(H)


def _make_spmm_kernel(scaled):
    def body(z_hbm, row_hbm, col_hbm, const_hbm, *rest):
        if scaled:
            (s2_hbm, out_hbm, row_v, gidx_v, buf0, buf1,
             acc, sem0, sem1) = rest
        else:
            out_hbm, row_v, gidx_v, buf0, buf1, acc, sem0, sem1 = rest
        c = lax.axis_index("c")
        s = lax.axis_index("s")

        base = c * NP

        pltpu.sync_copy(const_hbm.at[1], buf0)
        rbase = s * RPT
        for q in range(NFULL):
            pltpu.sync_copy(buf0, acc.at[pl.ds(rbase + q * K, K)])
        pltpu.sync_copy(buf0.at[pl.ds(0, REM)],
                        acc.at[pl.ds(rbase + NFULL * K, REM)])
        plsc.subcore_barrier()

        for half in range(2):
            cb = s * CHT_MM + half * CHT2
            pltpu.sync_copy(row_hbm.at[pl.ds(cb, CHT2)], row_v)
            pltpu.sync_copy(col_hbm.at[pl.ds(cb, CHT2)], gidx_v)

            @pl.loop(0, CHT2)
            def _(j):
                for i in range(K // 16):
                    sl = pl.ds(i * 16, 16)
                    gidx_v[j, sl] = gidx_v[j, sl] + base

            pltpu.async_copy(z_hbm.at[gidx_v.at[0]], buf0, sem0)
            pltpu.async_copy(z_hbm.at[gidx_v.at[1]], buf1, sem1)

            @pl.loop(0, CHT2, step=2)
            def _(g):
                for b, (buf, sem) in enumerate(((buf0, sem0), (buf1, sem1))):
                    j = g + b
                    pltpu.make_async_copy(z_hbm.at[gidx_v.at[j]], buf,
                                          sem).wait()
                    pltpu.sync_copy(buf, acc.at[row_v.at[j]], add=True)
                    jn = j + 2

                    @pl.when(jn < CHT2)
                    def _():
                        pltpu.async_copy(z_hbm.at[gidx_v.at[jn]], buf, sem)

        plsc.subcore_barrier()

        def flush(off, nrows, src_slice):
            pltpu.sync_copy(acc.at[pl.ds(off, nrows)], src_slice)
            if scaled:
                pltpu.sync_copy(s2_hbm.at[pl.ds(off, nrows)],
                                buf1.at[pl.ds(0, nrows)])

                @pl.loop(0, nrows)
                def _(r):
                    for i in range(H // 16):
                        sl = pl.ds(i * 16, 16)
                        buf0[r, sl] = buf0[r, sl] * buf1[r, sl]
            pltpu.sync_copy(src_slice, out_hbm.at[c, pl.ds(off, nrows)])

        for q in range(NFULL):
            flush(rbase + q * K, K, buf0)
        flush(rbase + NFULL * K, REM, buf0.at[pl.ds(0, REM)])

    scratch = [
        pltpu.VMEM((CHT2, K), jnp.int32),
        pltpu.VMEM((CHT2, K), jnp.int32),
        pltpu.VMEM((K, H), jnp.float32),
        pltpu.VMEM((K, H), jnp.float32),
        pltpu.VMEM_SHARED((NP, H), jnp.float32),
        pltpu.SemaphoreType.DMA,
        pltpu.SemaphoreType.DMA,
    ]
    return functools.partial(
        pl.kernel,
        out_type=jax.ShapeDtypeStruct((NC, NP, H), jnp.float32),
        mesh=_mesh,
        scratch_types=scratch,
    )(body)


_spmm_kernel = _make_spmm_kernel(scaled=False)
_spmm_scaled_kernel = _make_spmm_kernel(scaled=True)


def _rsqrt_deg(deg_ref):
    deg = deg_ref[0] + deg_ref[1]
    return lax.rsqrt(deg + EPS)


def _scale_in_body(deg_ref, x_ref, out_ref):
    deg = deg_ref[0] + deg_ref[1]
    sinv = lax.rsqrt(deg + EPS)
    out_ref[0] = sinv * x_ref[:, :H]
    out_ref[1] = sinv * x_ref[:, H:]


def _scale_mid_body(deg_ref, z_ref, out_ref):
    deg = deg_ref[0] + deg_ref[1]
    s2 = 1.0 / (deg + EPS)
    out_ref[0] = z_ref[0] * s2
    out_ref[1] = z_ref[1] * s2


def _final_body(deg_ref, z_ref, x_ref, out_ref):
    sinv = _rsqrt_deg(deg_ref)[:N]
    out_ref[:, :H] = x_ref[:, :H] + sinv * z_ref[0, :N, :]
    out_ref[:, H:] = x_ref[:, H:] + sinv * z_ref[1, :N, :]


def kernel(x, edge_index):
    row = edge_index[0].astype(jnp.int32)
    col = edge_index[1].astype(jnp.int32)
    pad = EP - E
    rowp = jnp.concatenate([row, jnp.full((pad,), N, jnp.int32)]).reshape(NCHUNK, K)
    colp = jnp.concatenate([col, jnp.full((pad,), N, jnp.int32)]).reshape(NCHUNK, K)
    xp = jnp.zeros((NP, D), x.dtype).at[:N].set(x)
    const = jnp.stack([jnp.ones((K, H), jnp.float32),
                       jnp.zeros((K, H), jnp.float32)])

    deg2 = _deg_kernel(rowp, const)

    z0 = pl.pallas_call(
        _scale_in_body,
        out_shape=jax.ShapeDtypeStruct((NC, NP, H), jnp.float32),
    )(deg2, xp)

    z1 = _spmm_kernel(z0.reshape(NC * NP, H), rowp, colp, const)

    z1s = pl.pallas_call(
        _scale_mid_body,
        out_shape=jax.ShapeDtypeStruct((NC, NP, H), jnp.float32),
    )(deg2, z1)

    z2 = _spmm_kernel(z1s.reshape(NC * NP, H), rowp, colp, const)

    out = pl.pallas_call(
        _final_body,
        out_shape=jax.ShapeDtypeStruct((N, D), jnp.float32),
    )(deg2, z2, x)
    return out


# speedup vs baseline: 1.0725x; 1.0223x over previous
import functools

import jax
import jax.numpy as jnp
from jax import lax
from jax.experimental import pallas as pl
from jax.experimental.pallas import tpu as pltpu
from jax.experimental.pallas import tpu_sc as plsc

N = 10000
E = 160000
D = 256
H = 128
NC = 2
NS = 16
K = 128
EP = 163840
NCHUNK = EP // K
CHT_MM = NCHUNK // NS
CHT2 = CHT_MM // 2
CHT_HIST = NCHUNK // (NC * NS)
NP = 10112
RPT = NP // NS
NFULL = RPT // K
REM = RPT % K
EPS = 1e-7

_mesh = plsc.VectorSubcoreMesh(core_axis_name="c", subcore_axis_name="s")


def _make_deg_kernel(W):
    @functools.partial(
        pl.kernel,
        out_type=jax.ShapeDtypeStruct((NC, NP, W), jnp.float32),
        mesh=_mesh,
        scratch_types=[
            pltpu.VMEM((CHT_HIST, K), jnp.int32),
            pltpu.VMEM((K, W), jnp.float32),
            pltpu.VMEM((K, W), jnp.float32),
            pltpu.VMEM_SHARED((NP, W), jnp.float32),
        ],
    )
    def deg_kernel(row_hbm, out_hbm, row_v, ones_v, stage_v, acc):
        c = lax.axis_index("c")
        s = lax.axis_index("s")
        w = c * NS + s
        pltpu.sync_copy(row_hbm.at[pl.ds(w * CHT_HIST, CHT_HIST)], row_v)

        ones16 = jnp.ones((16,), jnp.float32)
        zeros16 = jnp.zeros((16,), jnp.float32)

        @pl.loop(0, K)
        def _(r):
            for i in range(W // 16):
                ones_v[r, pl.ds(i * 16, 16)] = ones16
                stage_v[r, pl.ds(i * 16, 16)] = zeros16

        rbase = s * RPT
        for q in range(NFULL):
            pltpu.sync_copy(stage_v, acc.at[pl.ds(rbase + q * K, K)])
        pltpu.sync_copy(stage_v.at[pl.ds(0, REM)],
                        acc.at[pl.ds(rbase + NFULL * K, REM)])
        plsc.subcore_barrier()

        @pl.loop(0, CHT_HIST)
        def _(j):
            pltpu.sync_copy(ones_v, acc.at[row_v.at[j]], add=True)

        plsc.subcore_barrier()

        for q in range(NFULL):
            pltpu.sync_copy(acc.at[pl.ds(rbase + q * K, K)], stage_v)
            pltpu.sync_copy(stage_v, out_hbm.at[c, pl.ds(rbase + q * K, K)])
        pltpu.sync_copy(acc.at[pl.ds(rbase + NFULL * K, REM)],
                        stage_v.at[pl.ds(0, REM)])
        pltpu.sync_copy(stage_v.at[pl.ds(0, REM)],
                        out_hbm.at[c, pl.ds(rbase + NFULL * K, REM)])

    return deg_kernel


_deg_kernel = _make_deg_kernel(H)


def _make_spmm_kernel(scaled):
    def body(z_hbm, row_hbm, col_hbm, *rest):
        if scaled:
            (s2_hbm, out_hbm, row_v, gidx_v, buf0, buf1,
             acc, sem0, sem1) = rest
        else:
            out_hbm, row_v, gidx_v, buf0, buf1, acc, sem0, sem1 = rest
        c = lax.axis_index("c")
        s = lax.axis_index("s")

        base = c * NP

        zeros16 = jnp.zeros((16,), jnp.float32)

        @pl.loop(0, K)
        def _(r):
            for i in range(H // 16):
                buf0[r, pl.ds(i * 16, 16)] = zeros16

        rbase = s * RPT
        for q in range(NFULL):
            pltpu.sync_copy(buf0, acc.at[pl.ds(rbase + q * K, K)])
        pltpu.sync_copy(buf0.at[pl.ds(0, REM)],
                        acc.at[pl.ds(rbase + NFULL * K, REM)])
        plsc.subcore_barrier()

        for half in range(2):
            cb = s * CHT_MM + half * CHT2
            pltpu.sync_copy(row_hbm.at[pl.ds(cb, CHT2)], row_v)
            pltpu.sync_copy(col_hbm.at[pl.ds(cb, CHT2)], gidx_v)

            @pl.loop(0, CHT2)
            def _(j):
                for i in range(K // 16):
                    sl = pl.ds(i * 16, 16)
                    gidx_v[j, sl] = gidx_v[j, sl] + base

            pltpu.async_copy(z_hbm.at[gidx_v.at[0]], buf0, sem0)
            pltpu.async_copy(z_hbm.at[gidx_v.at[1]], buf1, sem1)

            @pl.loop(0, CHT2, step=2)
            def _(g):
                for b, (buf, sem) in enumerate(((buf0, sem0), (buf1, sem1))):
                    j = g + b
                    pltpu.make_async_copy(z_hbm.at[gidx_v.at[j]], buf,
                                          sem).wait()
                    pltpu.sync_copy(buf, acc.at[row_v.at[j]], add=True)
                    jn = j + 2

                    @pl.when(jn < CHT2)
                    def _():
                        pltpu.async_copy(z_hbm.at[gidx_v.at[jn]], buf, sem)

        plsc.subcore_barrier()

        def flush(off, nrows, src_slice):
            pltpu.sync_copy(acc.at[pl.ds(off, nrows)], src_slice)
            if scaled:
                pltpu.sync_copy(s2_hbm.at[pl.ds(off, nrows)],
                                buf1.at[pl.ds(0, nrows)])

                @pl.loop(0, nrows)
                def _(r):
                    for i in range(H // 16):
                        sl = pl.ds(i * 16, 16)
                        buf0[r, sl] = buf0[r, sl] * buf1[r, sl]
            pltpu.sync_copy(src_slice, out_hbm.at[c, pl.ds(off, nrows)])

        for q in range(NFULL):
            flush(rbase + q * K, K, buf0)
        flush(rbase + NFULL * K, REM, buf0.at[pl.ds(0, REM)])

    scratch = [
        pltpu.VMEM((CHT2, K), jnp.int32),
        pltpu.VMEM((CHT2, K), jnp.int32),
        pltpu.VMEM((K, H), jnp.float32),
        pltpu.VMEM((K, H), jnp.float32),
        pltpu.VMEM_SHARED((NP, H), jnp.float32),
        pltpu.SemaphoreType.DMA,
        pltpu.SemaphoreType.DMA,
    ]
    return functools.partial(
        pl.kernel,
        out_type=jax.ShapeDtypeStruct((NC, NP, H), jnp.float32),
        mesh=_mesh,
        scratch_types=scratch,
    )(body)


_spmm_kernel = _make_spmm_kernel(scaled=False)
_spmm_scaled_kernel = _make_spmm_kernel(scaled=True)


def _rsqrt_deg(deg_ref):
    deg = deg_ref[0] + deg_ref[1]
    return lax.rsqrt(deg + EPS)


def _scale_in_body(deg_ref, x_ref, out_ref):
    deg = deg_ref[0] + deg_ref[1]
    sinv = lax.rsqrt(deg + EPS)
    out_ref[0] = sinv * x_ref[:, :H]
    out_ref[1] = sinv * x_ref[:, H:]


def _scale_mid_body(deg_ref, z_ref, out_ref):
    deg = deg_ref[0] + deg_ref[1]
    s2 = 1.0 / (deg + EPS)
    out_ref[0] = z_ref[0] * s2
    out_ref[1] = z_ref[1] * s2


def _final_body(deg_ref, z_ref, x_ref, out_ref):
    sinv = _rsqrt_deg(deg_ref)[:N]
    out_ref[:, :H] = x_ref[:, :H] + sinv * z_ref[0, :N, :]
    out_ref[:, H:] = x_ref[:, H:] + sinv * z_ref[1, :N, :]


def kernel(x, edge_index):
    row = edge_index[0].astype(jnp.int32)
    col = edge_index[1].astype(jnp.int32)
    pad = EP - E
    rowp = jnp.concatenate([row, jnp.full((pad,), N, jnp.int32)]).reshape(NCHUNK, K)
    colp = jnp.concatenate([col, jnp.full((pad,), N, jnp.int32)]).reshape(NCHUNK, K)
    xp = jnp.zeros((NP, D), x.dtype).at[:N].set(x)

    deg2 = _deg_kernel(rowp)

    z0 = pl.pallas_call(
        _scale_in_body,
        out_shape=jax.ShapeDtypeStruct((NC, NP, H), jnp.float32),
    )(deg2, xp)

    z1 = _spmm_kernel(z0.reshape(NC * NP, H), rowp, colp)

    z1s = pl.pallas_call(
        _scale_mid_body,
        out_shape=jax.ShapeDtypeStruct((NC, NP, H), jnp.float32),
    )(deg2, z1)

    z2 = _spmm_kernel(z1s.reshape(NC * NP, H), rowp, colp)

    out = pl.pallas_call(
        _final_body,
        out_shape=jax.ShapeDtypeStruct((N, D), jnp.float32),
    )(deg2, z2, x)
    return out
